# Initial kernel scaffold; baseline (speedup 1.0000x reference)
import jax
import jax.numpy as jnp
from jax.experimental import pallas as pl


def kernel(idx, table):
    raise NotImplementedError("write your pallas kernel here")


# speedup vs baseline: 1.3942x; 1.3942x over previous
import functools

import jax
import jax.numpy as jnp
from jax import lax
from jax.experimental import pallas as pl
from jax.experimental.pallas import tpu as pltpu
from jax.experimental.pallas import tpu_sc as plsc


def _make_gather(n, V, D, NC, NS):
    NW = NC * NS
    b_per_w = n // NW
    CH = 32
    n_chunks = b_per_w // CH
    mesh = plsc.VectorSubcoreMesh(core_axis_name="c", subcore_axis_name="s")

    @functools.partial(
        pl.kernel,
        mesh=mesh,
        out_type=jax.ShapeDtypeStruct((n, D), jnp.float32),
        scratch_types=[
            pltpu.VMEM((b_per_w,), jnp.int32),
            pltpu.VMEM((CH, D), jnp.float32),
            pltpu.SemaphoreType.DMA,
        ],
    )
    def gather_kernel(idx_hbm, table_hbm, out_hbm, idx_v, rows_v, gsem):
        wid = lax.axis_index("s") * NC + lax.axis_index("c")
        base = wid * b_per_w
        pltpu.sync_copy(idx_hbm.at[pl.ds(base, b_per_w)], idx_v)
        for c in range(n_chunks):
            pltpu.async_copy(
                table_hbm.at[idx_v.at[pl.ds(c * CH, CH)]], rows_v, gsem
            ).wait()
            pltpu.sync_copy(rows_v, out_hbm.at[pl.ds(base + c * CH, CH)])

    return gather_kernel


def kernel(idx, table):
    B, S = idx.shape
    V, D = table.shape
    n = B * S
    flat_idx = idx.reshape(n).astype(jnp.int32)
    info = plsc.get_sparse_core_info()
    out = _make_gather(n, V, D, info.num_cores, info.num_subcores)(flat_idx, table)
    return out.reshape(B, S, D)


# speedup vs baseline: 1.5167x; 1.0879x over previous
import functools

import jax
import jax.numpy as jnp
from jax import lax
from jax.experimental import pallas as pl
from jax.experimental.pallas import tpu as pltpu
from jax.experimental.pallas import tpu_sc as plsc


def _make_gather(n, V, D, NC, NS):
    NW = NC * NS
    b_per_w = n // NW
    CH = 32
    n_chunks = b_per_w // CH
    mesh = plsc.VectorSubcoreMesh(core_axis_name="c", subcore_axis_name="s")

    @functools.partial(
        pl.kernel,
        mesh=mesh,
        out_type=jax.ShapeDtypeStruct((n, D), jnp.float32),
        scratch_types=[
            pltpu.VMEM((b_per_w,), jnp.int32),
            pltpu.VMEM((2, CH, D), jnp.float32),
            pltpu.SemaphoreType.DMA,
            pltpu.SemaphoreType.DMA,
            pltpu.SemaphoreType.DMA,
            pltpu.SemaphoreType.DMA,
        ],
    )
    def gather_kernel(idx_hbm, table_hbm, out_hbm, idx_v, rows_v, g0, g1, s0, s1):
        wid = lax.axis_index("s") * NC + lax.axis_index("c")
        base = wid * b_per_w
        gsem = (g0, g1)
        ssem = (s0, s1)
        pltpu.sync_copy(idx_hbm.at[pl.ds(base, b_per_w)], idx_v)

        def start_gather(c):
            return pltpu.async_copy(
                table_hbm.at[idx_v.at[pl.ds(c * CH, CH)]],
                rows_v.at[c % 2],
                gsem[c % 2],
            )

        def start_put(c):
            return pltpu.async_copy(
                rows_v.at[c % 2],
                out_hbm.at[pl.ds(base + c * CH, CH)],
                ssem[c % 2],
            )

        gathers = [None] * n_chunks
        puts = [None] * n_chunks
        gathers[0] = start_gather(0)
        for c in range(n_chunks):
            if c + 1 < n_chunks:
                if c >= 1:
                    puts[c - 1].wait()
                gathers[c + 1] = start_gather(c + 1)
            gathers[c].wait()
            puts[c] = start_put(c)
        puts[n_chunks - 2].wait()
        puts[n_chunks - 1].wait()

    return gather_kernel


def kernel(idx, table):
    B, S = idx.shape
    V, D = table.shape
    n = B * S
    flat_idx = idx.reshape(n).astype(jnp.int32)
    info = plsc.get_sparse_core_info()
    out = _make_gather(n, V, D, info.num_cores, info.num_subcores)(flat_idx, table)
    return out.reshape(B, S, D)


# speedup vs baseline: 1.5615x; 1.0295x over previous
import functools

import jax
import jax.numpy as jnp
from jax import lax
from jax.experimental import pallas as pl
from jax.experimental.pallas import tpu as pltpu
from jax.experimental.pallas import tpu_sc as plsc


def _make_gather(n, V, D, NC, NS):
    NW = NC * NS
    b_per_w = n // NW
    CH = 32
    n_chunks = b_per_w // CH
    mesh = plsc.VectorSubcoreMesh(core_axis_name="c", subcore_axis_name="s")

    NBUF = 3
    scratch = [
        pltpu.VMEM((b_per_w,), jnp.int32),
        pltpu.VMEM((NBUF, CH, D), jnp.float32),
    ]
    scratch += [pltpu.SemaphoreType.DMA] * (2 * NBUF)

    @functools.partial(
        pl.kernel,
        mesh=mesh,
        out_type=jax.ShapeDtypeStruct((n, D), jnp.float32),
        scratch_types=scratch,
    )
    def gather_kernel(idx_hbm, table_hbm, out_hbm, idx_v, rows_v, *sems):
        wid = lax.axis_index("s") * NC + lax.axis_index("c")
        base = wid * b_per_w
        gsem = sems[:NBUF]
        ssem = sems[NBUF:]
        pltpu.sync_copy(idx_hbm.at[pl.ds(base, b_per_w)], idx_v)

        def start_gather(c):
            return pltpu.async_copy(
                table_hbm.at[idx_v.at[pl.ds(c * CH, CH)]],
                rows_v.at[c % NBUF],
                gsem[c % NBUF],
            )

        def start_put(c):
            return pltpu.async_copy(
                rows_v.at[c % NBUF],
                out_hbm.at[pl.ds(base + c * CH, CH)],
                ssem[c % NBUF],
            )

        gathers = [None] * n_chunks
        puts = [None] * n_chunks
        for c in range(min(NBUF, n_chunks)):
            gathers[c] = start_gather(c)
        for c in range(n_chunks):
            prev = c - 1
            if prev >= 0 and prev + NBUF < n_chunks:
                puts[prev].wait()
                gathers[prev + NBUF] = start_gather(prev + NBUF)
            gathers[c].wait()
            puts[c] = start_put(c)
        for c in range(n_chunks):
            if puts[c] is not None and c + NBUF >= n_chunks:
                puts[c].wait()

    return gather_kernel


def kernel(idx, table):
    B, S = idx.shape
    V, D = table.shape
    n = B * S
    flat_idx = idx.reshape(n).astype(jnp.int32)
    info = plsc.get_sparse_core_info()
    out = _make_gather(n, V, D, info.num_cores, info.num_subcores)(flat_idx, table)
    return out.reshape(B, S, D)
